# baseline (speedup 1.0000x reference)
import functools

import jax
import jax.numpy as jnp
from jax import lax
from jax.experimental import pallas as pl
from jax.experimental.pallas import tpu as pltpu
from jax.experimental.pallas import tpu_sc as plsc

N = 10000
E = 320000
D = 128
HEADS = 8
HID = 16
NCLS = 64

NCORES = 2
NSUB = 16
NW = NCORES * NSUB
EPW = E // NW
EB = 80
NCHUNK = EPW // EB
ACCW = D + 16
RPT = N // NSUB
RZ = 125

ROWB = 1000
GRID = N // ROWB


def _dense_in_body(x_ref, w_ref, ascat_ref, h_ref, asd_ref):
    h = jnp.dot(x_ref[...], w_ref[...], preferred_element_type=jnp.float32)
    h_ref[...] = h
    asd_ref[...] = jnp.dot(h, ascat_ref[...], preferred_element_type=jnp.float32)


def _dense_in(x, w, ascat):
    return pl.pallas_call(
        _dense_in_body,
        grid=(GRID,),
        in_specs=[
            pl.BlockSpec((ROWB, D), lambda i: (i, 0)),
            pl.BlockSpec((D, D), lambda i: (0, 0)),
            pl.BlockSpec((D, 16), lambda i: (0, 0)),
        ],
        out_specs=[
            pl.BlockSpec((ROWB, D), lambda i: (i, 0)),
            pl.BlockSpec((ROWB, 16), lambda i: (i, 0)),
        ],
        out_shape=[
            jax.ShapeDtypeStruct((N, D), jnp.float32),
            jax.ShapeDtypeStruct((N, 16), jnp.float32),
        ],
    )(x, w, ascat)


def _combine(msgs, dens, h, asd, exp8, bias, g, b):
    comb_m = msgs[0] + msgs[1]
    d8 = dens[0][:, :HEADS] + dens[1][:, :HEADS]
    ss = asd[:, :HEADS] + asd[:, HEADS:2 * HEADS]
    wself = jnp.exp(jnp.maximum(ss, 0.2 * ss))
    wself128 = jnp.dot(wself, exp8, preferred_element_type=jnp.float32)
    num = comb_m + wself128 * h
    den = jnp.dot(d8 + wself, exp8, preferred_element_type=jnp.float32)
    o = num / (den + 1e-16) + bias
    m = o.mean(-1, keepdims=True)
    v = ((o - m) ** 2).mean(-1, keepdims=True)
    return jax.nn.relu((o - m) / jnp.sqrt(v + 1e-5) * g + b)


def _dense_mid_body(msgs_ref, dens_ref, h_ref, asd_ref, exp8_ref, b1_ref,
                    g_ref, bln_ref, w2_ref, ascat2_ref, h2_ref, asd2_ref):
    y = _combine(msgs_ref[...], dens_ref[...], h_ref[...], asd_ref[...],
                 exp8_ref[...], b1_ref[...], g_ref[...], bln_ref[...])
    h2 = jnp.dot(y, w2_ref[...], preferred_element_type=jnp.float32)
    h2_ref[...] = h2
    asd2_ref[...] = jnp.dot(h2, ascat2_ref[...], preferred_element_type=jnp.float32)


def _dense_mid(msgs, dens, h, asd, exp8, b1, g, bln, w2, ascat2):
    return pl.pallas_call(
        _dense_mid_body,
        grid=(GRID,),
        in_specs=[
            pl.BlockSpec((2, ROWB, D), lambda i: (0, i, 0)),
            pl.BlockSpec((2, ROWB, 16), lambda i: (0, i, 0)),
            pl.BlockSpec((ROWB, D), lambda i: (i, 0)),
            pl.BlockSpec((ROWB, 16), lambda i: (i, 0)),
            pl.BlockSpec((HEADS, D), lambda i: (0, 0)),
            pl.BlockSpec((1, D), lambda i: (0, 0)),
            pl.BlockSpec((1, D), lambda i: (0, 0)),
            pl.BlockSpec((1, D), lambda i: (0, 0)),
            pl.BlockSpec((D, D), lambda i: (0, 0)),
            pl.BlockSpec((D, 16), lambda i: (0, 0)),
        ],
        out_specs=[
            pl.BlockSpec((ROWB, D), lambda i: (i, 0)),
            pl.BlockSpec((ROWB, 16), lambda i: (i, 0)),
        ],
        out_shape=[
            jax.ShapeDtypeStruct((N, D), jnp.float32),
            jax.ShapeDtypeStruct((N, 16), jnp.float32),
        ],
    )(msgs, dens, h, asd, exp8, b1, g, bln, w2, ascat2)


def _dense_out_body(msgs_ref, dens_ref, h_ref, asd_ref, exp8_ref, b2_ref,
                    g_ref, bln_ref, wout_ref, bout_ref, out_ref):
    y = _combine(msgs_ref[...], dens_ref[...], h_ref[...], asd_ref[...],
                 exp8_ref[...], b2_ref[...], g_ref[...], bln_ref[...])
    out_ref[...] = jnp.dot(y, wout_ref[...],
                           preferred_element_type=jnp.float32) + bout_ref[...]


def _dense_out(msgs, dens, h, asd, exp8, b2, g, bln, wout, bout):
    return pl.pallas_call(
        _dense_out_body,
        grid=(GRID,),
        in_specs=[
            pl.BlockSpec((2, ROWB, D), lambda i: (0, i, 0)),
            pl.BlockSpec((2, ROWB, 16), lambda i: (0, i, 0)),
            pl.BlockSpec((ROWB, D), lambda i: (i, 0)),
            pl.BlockSpec((ROWB, 16), lambda i: (i, 0)),
            pl.BlockSpec((HEADS, D), lambda i: (0, 0)),
            pl.BlockSpec((1, D), lambda i: (0, 0)),
            pl.BlockSpec((1, D), lambda i: (0, 0)),
            pl.BlockSpec((1, D), lambda i: (0, 0)),
            pl.BlockSpec((D, NCLS), lambda i: (0, 0)),
            pl.BlockSpec((1, NCLS), lambda i: (0, 0)),
        ],
        out_specs=[pl.BlockSpec((ROWB, NCLS), lambda i: (i, 0))],
        out_shape=[jax.ShapeDtypeStruct((N, NCLS), jnp.float32)],
    )(msgs, dens, h, asd, exp8, b2, g, bln, wout, bout)


def _edge_body(src_hbm, dst_hbm, asd_hbm, dsa_hbm, h_hbm, out_hbm,
               src_v0, src_v1, dst_v0, dst_v1, as_v0, as_v1, ad_v0, ad_v1,
               h_v0, h_v1, msg_v, acc, sem0, sem1):
    src_v = (src_v0, src_v1)
    dst_v = (dst_v0, dst_v1)
    as_v = (as_v0, as_v1)
    ad_v = (ad_v0, ad_v1)
    h_v = (h_v0, h_v1)
    sems = (sem0, sem1)
    c = lax.axis_index("c")
    s = lax.axis_index("s")
    wid = s * NCORES + c

    zeros16 = jnp.zeros((16,), jnp.float32)

    def zrow(r, carry):
        for j in range(ACCW // 16):
            msg_v[r, pl.ds(j * 16, 16)] = zeros16
        return carry

    lax.fori_loop(0, EB, zrow, 0)
    base_r = s * RPT
    nfull = RPT // EB
    for k in range(nfull):
        pltpu.sync_copy(msg_v, acc.at[pl.ds(base_r + k * EB, EB)])
    rem = RPT - nfull * EB
    if rem:
        pltpu.sync_copy(msg_v.at[pl.ds(0, rem)],
                        acc.at[pl.ds(base_r + nfull * EB, rem)])
    plsc.subcore_barrier()

    def issue(ch, p):
        base = wid * EPW + ch * EB
        pltpu.sync_copy(src_hbm.at[pl.ds(base, EB)], src_v[p])
        pltpu.sync_copy(dst_hbm.at[pl.ds(base, EB)], dst_v[p])
        pltpu.async_copy(asd_hbm.at[src_v[p]], as_v[p], sems[p])
        pltpu.async_copy(dsa_hbm.at[dst_v[p]], ad_v[p], sems[p])
        pltpu.async_copy(h_hbm.at[src_v[p]], h_v[p], sems[p])

    def drain(p):
        pltpu.make_async_copy(asd_hbm.at[src_v[p]], as_v[p], sems[p]).wait()
        pltpu.make_async_copy(dsa_hbm.at[dst_v[p]], ad_v[p], sems[p]).wait()
        pltpu.make_async_copy(h_hbm.at[src_v[p]], h_v[p], sems[p]).wait()

    def compute(p):
        def edge(bi, ecarry):
            sv = as_v[p][bi, :] + ad_v[p][bi, :]
            w = jnp.exp(jnp.maximum(sv, 0.2 * sv))
            msg_v[bi, pl.ds(D, 16)] = w
            for hh in range(HEADS):
                wsp = lax.gather(
                    w, jnp.full((16, 1), hh, jnp.int32),
                    lax.GatherDimensionNumbers(
                        offset_dims=(), collapsed_slice_dims=(0,),
                        start_index_map=(0,)),
                    (1,), mode=lax.GatherScatterMode.PROMISE_IN_BOUNDS)
                msg_v[bi, pl.ds(hh * 16, 16)] = (
                    h_v[p][bi, pl.ds(hh * 16, 16)] * wsp)
            return ecarry

        lax.fori_loop(0, EB, edge, 0)
        pltpu.sync_copy(msg_v, acc.at[dst_v[p]], add=True)

    issue(0, 0)

    def chunk2(i, carry):
        ch = 2 * i
        issue(ch + 1, 1)
        drain(0)
        compute(0)
        issue(ch + 2, 0)
        drain(1)
        compute(1)
        return carry

    lax.fori_loop(0, (NCHUNK - 1) // 2, chunk2, 0)
    drain(0)
    compute(0)
    plsc.subcore_barrier()

    for k in range(nfull):
        rows = pl.ds(base_r + k * EB, EB)
        pltpu.sync_copy(acc.at[rows], out_hbm.at[c, rows])
    if rem:
        rows = pl.ds(base_r + nfull * EB, rem)
        pltpu.sync_copy(acc.at[rows], out_hbm.at[c, rows])


@functools.lru_cache(maxsize=1)
def _make_edge_kernel():
    return pl.kernel(
        _edge_body,
        out_type=jax.ShapeDtypeStruct((NCORES, N, ACCW), jnp.float32),
        mesh=plsc.VectorSubcoreMesh(core_axis_name="c", subcore_axis_name="s",
                                    num_cores=NCORES, num_subcores=NSUB),
        compiler_params=pltpu.CompilerParams(use_tc_tiling_on_sc=False),
        scratch_types=[
            pltpu.VMEM((EB,), jnp.int32),
            pltpu.VMEM((EB,), jnp.int32),
            pltpu.VMEM((EB,), jnp.int32),
            pltpu.VMEM((EB,), jnp.int32),
            pltpu.VMEM((EB, 16), jnp.float32),
            pltpu.VMEM((EB, 16), jnp.float32),
            pltpu.VMEM((EB, 16), jnp.float32),
            pltpu.VMEM((EB, 16), jnp.float32),
            pltpu.VMEM((EB, D), jnp.float32),
            pltpu.VMEM((EB, D), jnp.float32),
            pltpu.VMEM((EB, ACCW), jnp.float32),
            pltpu.VMEM_SHARED((N, ACCW), jnp.float32),
            pltpu.SemaphoreType.DMA,
            pltpu.SemaphoreType.DMA,
        ],
    )


def _edge_pass(src, dst, asd, h):
    dsa = jnp.concatenate([asd[:, HEADS:], asd[:, :HEADS]], axis=1)
    acc = _make_edge_kernel()(src, dst, asd, dsa, h)
    return acc[:, :, :D], acc[:, :, D:]


def _ascat(a_src, a_dst):
    mask = (jnp.arange(D)[:, None] // HID == jnp.arange(HEADS)[None, :])
    asm = jnp.where(mask, a_src.reshape(D)[:, None], 0.0)
    adm = jnp.where(mask, a_dst.reshape(D)[:, None], 0.0)
    return jnp.concatenate([asm, adm], axis=1)


def kernel(x, edge_index, W1, a_src1, a_dst1, b1, ln1_g, ln1_b,
           W2, a_src2, a_dst2, b2, ln2_g, ln2_b, W_out, b_out):
    src = edge_index[0]
    dst = edge_index[1]
    exp8 = (jnp.arange(HEADS)[:, None] == jnp.arange(D)[None, :] // HID)
    exp8 = exp8.astype(jnp.float32)

    h1, asd1 = _dense_in(x, W1, _ascat(a_src1, a_dst1))
    msgs1, dens1 = _edge_pass(src, dst, asd1, h1)
    h2, asd2 = _dense_mid(msgs1, dens1, h1, asd1, exp8,
                          b1.reshape(1, D), ln1_g.reshape(1, D),
                          ln1_b.reshape(1, D), W2, _ascat(a_src2, a_dst2))
    msgs2, dens2 = _edge_pass(src, dst, asd2, h2)
    out, = _dense_out(msgs2, dens2, h2, asd2, exp8,
                      b2.reshape(1, D), ln2_g.reshape(1, D),
                      ln2_b.reshape(1, D), W_out, b_out.reshape(1, NCLS))
    return out


# speedup vs baseline: 2.2865x; 2.2865x over previous
import functools

import jax
import jax.numpy as jnp
from jax import lax
from jax.experimental import pallas as pl
from jax.experimental.pallas import tpu as pltpu
from jax.experimental.pallas import tpu_sc as plsc

N = 10000
E = 320000
D = 128
HEADS = 8
HID = 16
NCLS = 64

NCORES = 2
NSUB = 16
NW = NCORES * NSUB
EPW = E // NW
EB = 80
NCHUNK = EPW // EB
ACCW = D + 16
RPT = N // NSUB
RZ = 125

ROWB = 1000
GRID = N // ROWB


def _dense_in_body(x_ref, w_ref, ascat_ref, h_ref, asd_ref):
    h = jnp.dot(x_ref[...], w_ref[...], preferred_element_type=jnp.float32)
    h_ref[...] = h
    asd_ref[...] = jnp.dot(h, ascat_ref[...], preferred_element_type=jnp.float32)


def _dense_in(x, w, ascat):
    return pl.pallas_call(
        _dense_in_body,
        grid=(GRID,),
        in_specs=[
            pl.BlockSpec((ROWB, D), lambda i: (i, 0)),
            pl.BlockSpec((D, D), lambda i: (0, 0)),
            pl.BlockSpec((D, 16), lambda i: (0, 0)),
        ],
        out_specs=[
            pl.BlockSpec((ROWB, D), lambda i: (i, 0)),
            pl.BlockSpec((ROWB, 16), lambda i: (i, 0)),
        ],
        out_shape=[
            jax.ShapeDtypeStruct((N, D), jnp.float32),
            jax.ShapeDtypeStruct((N, 16), jnp.float32),
        ],
    )(x, w, ascat)


def _combine(msgs, dens, h, asd, exp8, bias, g, b):
    comb_m = msgs[0] + msgs[1]
    d8 = dens[0][:, :HEADS] + dens[1][:, :HEADS]
    ss = asd[:, :HEADS] + asd[:, HEADS:2 * HEADS]
    wself = jnp.exp(jnp.maximum(ss, 0.2 * ss))
    wself128 = jnp.dot(wself, exp8, preferred_element_type=jnp.float32)
    num = comb_m + wself128 * h
    den = jnp.dot(d8 + wself, exp8, preferred_element_type=jnp.float32)
    o = num / (den + 1e-16) + bias
    m = o.mean(-1, keepdims=True)
    v = ((o - m) ** 2).mean(-1, keepdims=True)
    return jax.nn.relu((o - m) / jnp.sqrt(v + 1e-5) * g + b)


def _dense_mid_body(msgs_ref, dens_ref, h_ref, asd_ref, exp8_ref, b1_ref,
                    g_ref, bln_ref, w2_ref, ascat2_ref, h2_ref, asd2_ref):
    y = _combine(msgs_ref[...], dens_ref[...], h_ref[...], asd_ref[...],
                 exp8_ref[...], b1_ref[...], g_ref[...], bln_ref[...])
    h2 = jnp.dot(y, w2_ref[...], preferred_element_type=jnp.float32)
    h2_ref[...] = h2
    asd2_ref[...] = jnp.dot(h2, ascat2_ref[...], preferred_element_type=jnp.float32)


def _dense_mid(msgs, dens, h, asd, exp8, b1, g, bln, w2, ascat2):
    return pl.pallas_call(
        _dense_mid_body,
        grid=(GRID,),
        in_specs=[
            pl.BlockSpec((2, ROWB, D), lambda i: (0, i, 0)),
            pl.BlockSpec((2, ROWB, 16), lambda i: (0, i, 0)),
            pl.BlockSpec((ROWB, D), lambda i: (i, 0)),
            pl.BlockSpec((ROWB, 16), lambda i: (i, 0)),
            pl.BlockSpec((HEADS, D), lambda i: (0, 0)),
            pl.BlockSpec((1, D), lambda i: (0, 0)),
            pl.BlockSpec((1, D), lambda i: (0, 0)),
            pl.BlockSpec((1, D), lambda i: (0, 0)),
            pl.BlockSpec((D, D), lambda i: (0, 0)),
            pl.BlockSpec((D, 16), lambda i: (0, 0)),
        ],
        out_specs=[
            pl.BlockSpec((ROWB, D), lambda i: (i, 0)),
            pl.BlockSpec((ROWB, 16), lambda i: (i, 0)),
        ],
        out_shape=[
            jax.ShapeDtypeStruct((N, D), jnp.float32),
            jax.ShapeDtypeStruct((N, 16), jnp.float32),
        ],
    )(msgs, dens, h, asd, exp8, b1, g, bln, w2, ascat2)


def _dense_out_body(msgs_ref, dens_ref, h_ref, asd_ref, exp8_ref, b2_ref,
                    g_ref, bln_ref, wout_ref, bout_ref, out_ref):
    y = _combine(msgs_ref[...], dens_ref[...], h_ref[...], asd_ref[...],
                 exp8_ref[...], b2_ref[...], g_ref[...], bln_ref[...])
    out_ref[...] = jnp.dot(y, wout_ref[...],
                           preferred_element_type=jnp.float32) + bout_ref[...]


def _dense_out(msgs, dens, h, asd, exp8, b2, g, bln, wout, bout):
    return pl.pallas_call(
        _dense_out_body,
        grid=(GRID,),
        in_specs=[
            pl.BlockSpec((2, ROWB, D), lambda i: (0, i, 0)),
            pl.BlockSpec((2, ROWB, 16), lambda i: (0, i, 0)),
            pl.BlockSpec((ROWB, D), lambda i: (i, 0)),
            pl.BlockSpec((ROWB, 16), lambda i: (i, 0)),
            pl.BlockSpec((HEADS, D), lambda i: (0, 0)),
            pl.BlockSpec((1, D), lambda i: (0, 0)),
            pl.BlockSpec((1, D), lambda i: (0, 0)),
            pl.BlockSpec((1, D), lambda i: (0, 0)),
            pl.BlockSpec((D, NCLS), lambda i: (0, 0)),
            pl.BlockSpec((1, NCLS), lambda i: (0, 0)),
        ],
        out_specs=[pl.BlockSpec((ROWB, NCLS), lambda i: (i, 0))],
        out_shape=[jax.ShapeDtypeStruct((N, NCLS), jnp.float32)],
    )(msgs, dens, h, asd, exp8, b2, g, bln, wout, bout)


def _edge_body(src_hbm, dst_hbm, asd_hbm, dsa_hbm, h_hbm, out_hbm,
               src_v0, src_v1, dst_v0, dst_v1, as_v0, as_v1, ad_v0, ad_v1,
               h_v0, h_v1, msg_v, acc, sem0, sem1):
    src_v = (src_v0, src_v1)
    dst_v = (dst_v0, dst_v1)
    as_v = (as_v0, as_v1)
    ad_v = (ad_v0, ad_v1)
    h_v = (h_v0, h_v1)
    sems = (sem0, sem1)
    c = lax.axis_index("c")
    s = lax.axis_index("s")
    wid = s * NCORES + c

    zeros16 = jnp.zeros((16,), jnp.float32)

    def zrow(r, carry):
        for j in range(ACCW // 16):
            msg_v[r, pl.ds(j * 16, 16)] = zeros16
        return carry

    lax.fori_loop(0, EB, zrow, 0)
    base_r = s * RPT
    nfull = RPT // EB
    for k in range(nfull):
        pltpu.sync_copy(msg_v, acc.at[pl.ds(base_r + k * EB, EB)])
    rem = RPT - nfull * EB
    if rem:
        pltpu.sync_copy(msg_v.at[pl.ds(0, rem)],
                        acc.at[pl.ds(base_r + nfull * EB, rem)])
    plsc.subcore_barrier()

    def issue(ch, p):
        base = wid * EPW + ch * EB
        pltpu.sync_copy(src_hbm.at[pl.ds(base, EB)], src_v[p])
        pltpu.sync_copy(dst_hbm.at[pl.ds(base, EB)], dst_v[p])
        pltpu.async_copy(asd_hbm.at[src_v[p]], as_v[p], sems[p])
        pltpu.async_copy(dsa_hbm.at[dst_v[p]], ad_v[p], sems[p])
        pltpu.async_copy(h_hbm.at[src_v[p]], h_v[p], sems[p])

    def drain(p):
        pltpu.make_async_copy(asd_hbm.at[src_v[p]], as_v[p], sems[p]).wait()
        pltpu.make_async_copy(dsa_hbm.at[dst_v[p]], ad_v[p], sems[p]).wait()
        pltpu.make_async_copy(h_hbm.at[src_v[p]], h_v[p], sems[p]).wait()

    def compute(p):
        @plsc.parallel_loop(0, EB, 1, unroll=4)
        def edge(bi):
            sv = as_v[p][bi, :] + ad_v[p][bi, :]
            w = jnp.exp(jnp.maximum(sv, 0.2 * sv))
            msg_v[bi, pl.ds(D, 16)] = w
            for hh in range(HEADS):
                wsp = lax.gather(
                    w, jnp.full((16, 1), hh, jnp.int32),
                    lax.GatherDimensionNumbers(
                        offset_dims=(), collapsed_slice_dims=(0,),
                        start_index_map=(0,)),
                    (1,), mode=lax.GatherScatterMode.PROMISE_IN_BOUNDS)
                msg_v[bi, pl.ds(hh * 16, 16)] = (
                    h_v[p][bi, pl.ds(hh * 16, 16)] * wsp)

        pltpu.sync_copy(msg_v, acc.at[dst_v[p]], add=True)

    issue(0, 0)

    def chunk2(i, carry):
        ch = 2 * i
        issue(ch + 1, 1)
        drain(0)
        compute(0)
        issue(ch + 2, 0)
        drain(1)
        compute(1)
        return carry

    lax.fori_loop(0, (NCHUNK - 1) // 2, chunk2, 0)
    drain(0)
    compute(0)
    plsc.subcore_barrier()

    for k in range(nfull):
        rows = pl.ds(base_r + k * EB, EB)
        pltpu.sync_copy(acc.at[rows], out_hbm.at[c, rows])
    if rem:
        rows = pl.ds(base_r + nfull * EB, rem)
        pltpu.sync_copy(acc.at[rows], out_hbm.at[c, rows])


@functools.lru_cache(maxsize=1)
def _make_edge_kernel():
    return pl.kernel(
        _edge_body,
        out_type=jax.ShapeDtypeStruct((NCORES, N, ACCW), jnp.float32),
        mesh=plsc.VectorSubcoreMesh(core_axis_name="c", subcore_axis_name="s",
                                    num_cores=NCORES, num_subcores=NSUB),
        compiler_params=pltpu.CompilerParams(use_tc_tiling_on_sc=False),
        scratch_types=[
            pltpu.VMEM((EB,), jnp.int32),
            pltpu.VMEM((EB,), jnp.int32),
            pltpu.VMEM((EB,), jnp.int32),
            pltpu.VMEM((EB,), jnp.int32),
            pltpu.VMEM((EB, 16), jnp.float32),
            pltpu.VMEM((EB, 16), jnp.float32),
            pltpu.VMEM((EB, 16), jnp.float32),
            pltpu.VMEM((EB, 16), jnp.float32),
            pltpu.VMEM((EB, D), jnp.float32),
            pltpu.VMEM((EB, D), jnp.float32),
            pltpu.VMEM((EB, ACCW), jnp.float32),
            pltpu.VMEM_SHARED((N, ACCW), jnp.float32),
            pltpu.SemaphoreType.DMA,
            pltpu.SemaphoreType.DMA,
        ],
    )


def _edge_pass(src, dst, asd, h):
    dsa = jnp.concatenate([asd[:, HEADS:], asd[:, :HEADS]], axis=1)
    acc = _make_edge_kernel()(src, dst, asd, dsa, h)
    return acc[:, :, :D], acc[:, :, D:]


def _ascat(a_src, a_dst):
    mask = (jnp.arange(D)[:, None] // HID == jnp.arange(HEADS)[None, :])
    asm = jnp.where(mask, a_src.reshape(D)[:, None], 0.0)
    adm = jnp.where(mask, a_dst.reshape(D)[:, None], 0.0)
    return jnp.concatenate([asm, adm], axis=1)


def kernel(x, edge_index, W1, a_src1, a_dst1, b1, ln1_g, ln1_b,
           W2, a_src2, a_dst2, b2, ln2_g, ln2_b, W_out, b_out):
    src = edge_index[0]
    dst = edge_index[1]
    exp8 = (jnp.arange(HEADS)[:, None] == jnp.arange(D)[None, :] // HID)
    exp8 = exp8.astype(jnp.float32)

    h1, asd1 = _dense_in(x, W1, _ascat(a_src1, a_dst1))
    msgs1, dens1 = _edge_pass(src, dst, asd1, h1)
    h2, asd2 = _dense_mid(msgs1, dens1, h1, asd1, exp8,
                          b1.reshape(1, D), ln1_g.reshape(1, D),
                          ln1_b.reshape(1, D), W2, _ascat(a_src2, a_dst2))
    msgs2, dens2 = _edge_pass(src, dst, asd2, h2)
    out, = _dense_out(msgs2, dens2, h2, asd2, exp8,
                      b2.reshape(1, D), ln2_g.reshape(1, D),
                      ln2_b.reshape(1, D), W_out, b_out.reshape(1, NCLS))
    return out


# speedup vs baseline: 2.7162x; 1.1879x over previous
import functools

import jax
import jax.numpy as jnp
from jax import lax
from jax.experimental import pallas as pl
from jax.experimental.pallas import tpu as pltpu
from jax.experimental.pallas import tpu_sc as plsc

N = 10000
E = 320000
D = 128
HEADS = 8
HID = 16
NCLS = 64

NCORES = 2
NSUB = 16
NW = NCORES * NSUB
EPW = E // NW
EB = 80
NCHUNK = EPW // EB
ACCW = D + 16
RPT = N // NSUB
RZ = 125

ROWB = 1000
GRID = N // ROWB


def _dense_in_body(x_ref, w_ref, ascat_ref, h_ref, asd_ref):
    h = jnp.dot(x_ref[...], w_ref[...], preferred_element_type=jnp.float32)
    h_ref[...] = h
    asd_ref[...] = jnp.dot(h, ascat_ref[...], preferred_element_type=jnp.float32)


def _dense_in(x, w, ascat):
    return pl.pallas_call(
        _dense_in_body,
        grid=(GRID,),
        in_specs=[
            pl.BlockSpec((ROWB, D), lambda i: (i, 0)),
            pl.BlockSpec((D, D), lambda i: (0, 0)),
            pl.BlockSpec((D, 16), lambda i: (0, 0)),
        ],
        out_specs=[
            pl.BlockSpec((ROWB, D), lambda i: (i, 0)),
            pl.BlockSpec((ROWB, 16), lambda i: (i, 0)),
        ],
        out_shape=[
            jax.ShapeDtypeStruct((N, D), jnp.float32),
            jax.ShapeDtypeStruct((N, 16), jnp.float32),
        ],
    )(x, w, ascat)


def _combine(msgs, dens, h, asd, exp8, bias, g, b):
    comb_m = msgs[0] + msgs[1]
    d8 = dens[0][:, :HEADS] + dens[1][:, :HEADS]
    ss = asd[:, :HEADS] + asd[:, HEADS:2 * HEADS]
    wself = jnp.exp(jnp.maximum(ss, 0.2 * ss))
    wself128 = jnp.dot(wself, exp8, preferred_element_type=jnp.float32)
    num = comb_m + wself128 * h
    den = jnp.dot(d8 + wself, exp8, preferred_element_type=jnp.float32)
    o = num / (den + 1e-16) + bias
    m = o.mean(-1, keepdims=True)
    v = ((o - m) ** 2).mean(-1, keepdims=True)
    return jax.nn.relu((o - m) / jnp.sqrt(v + 1e-5) * g + b)


def _dense_mid_body(msgs_ref, dens_ref, h_ref, asd_ref, exp8_ref, b1_ref,
                    g_ref, bln_ref, w2_ref, ascat2_ref, h2_ref, asd2_ref):
    y = _combine(msgs_ref[...], dens_ref[...], h_ref[...], asd_ref[...],
                 exp8_ref[...], b1_ref[...], g_ref[...], bln_ref[...])
    h2 = jnp.dot(y, w2_ref[...], preferred_element_type=jnp.float32)
    h2_ref[...] = h2
    asd2_ref[...] = jnp.dot(h2, ascat2_ref[...], preferred_element_type=jnp.float32)


def _dense_mid(msgs, dens, h, asd, exp8, b1, g, bln, w2, ascat2):
    return pl.pallas_call(
        _dense_mid_body,
        grid=(GRID,),
        in_specs=[
            pl.BlockSpec((2, ROWB, D), lambda i: (0, i, 0)),
            pl.BlockSpec((2, ROWB, 16), lambda i: (0, i, 0)),
            pl.BlockSpec((ROWB, D), lambda i: (i, 0)),
            pl.BlockSpec((ROWB, 16), lambda i: (i, 0)),
            pl.BlockSpec((HEADS, D), lambda i: (0, 0)),
            pl.BlockSpec((1, D), lambda i: (0, 0)),
            pl.BlockSpec((1, D), lambda i: (0, 0)),
            pl.BlockSpec((1, D), lambda i: (0, 0)),
            pl.BlockSpec((D, D), lambda i: (0, 0)),
            pl.BlockSpec((D, 16), lambda i: (0, 0)),
        ],
        out_specs=[
            pl.BlockSpec((ROWB, D), lambda i: (i, 0)),
            pl.BlockSpec((ROWB, 16), lambda i: (i, 0)),
        ],
        out_shape=[
            jax.ShapeDtypeStruct((N, D), jnp.float32),
            jax.ShapeDtypeStruct((N, 16), jnp.float32),
        ],
    )(msgs, dens, h, asd, exp8, b1, g, bln, w2, ascat2)


def _dense_out_body(msgs_ref, dens_ref, h_ref, asd_ref, exp8_ref, b2_ref,
                    g_ref, bln_ref, wout_ref, bout_ref, out_ref):
    y = _combine(msgs_ref[...], dens_ref[...], h_ref[...], asd_ref[...],
                 exp8_ref[...], b2_ref[...], g_ref[...], bln_ref[...])
    out_ref[...] = jnp.dot(y, wout_ref[...],
                           preferred_element_type=jnp.float32) + bout_ref[...]


def _dense_out(msgs, dens, h, asd, exp8, b2, g, bln, wout, bout):
    return pl.pallas_call(
        _dense_out_body,
        grid=(GRID,),
        in_specs=[
            pl.BlockSpec((2, ROWB, D), lambda i: (0, i, 0)),
            pl.BlockSpec((2, ROWB, 16), lambda i: (0, i, 0)),
            pl.BlockSpec((ROWB, D), lambda i: (i, 0)),
            pl.BlockSpec((ROWB, 16), lambda i: (i, 0)),
            pl.BlockSpec((HEADS, D), lambda i: (0, 0)),
            pl.BlockSpec((1, D), lambda i: (0, 0)),
            pl.BlockSpec((1, D), lambda i: (0, 0)),
            pl.BlockSpec((1, D), lambda i: (0, 0)),
            pl.BlockSpec((D, NCLS), lambda i: (0, 0)),
            pl.BlockSpec((1, NCLS), lambda i: (0, 0)),
        ],
        out_specs=[pl.BlockSpec((ROWB, NCLS), lambda i: (i, 0))],
        out_shape=[jax.ShapeDtypeStruct((N, NCLS), jnp.float32)],
    )(msgs, dens, h, asd, exp8, b2, g, bln, wout, bout)


def _edge_body(ei_hbm, asd_hbm, dsa_hbm, h_hbm, out_hbm,
               sd_v0, sd_v1, as_v0, as_v1, ad_v0, ad_v1,
               h_v0, h_v1, msg_v, acc, sem0, sem1):
    sd_v = (sd_v0, sd_v1)
    as_v = (as_v0, as_v1)
    ad_v = (ad_v0, ad_v1)
    h_v = (h_v0, h_v1)
    sems = (sem0, sem1)
    c = lax.axis_index("c")
    s = lax.axis_index("s")
    wid = s * NCORES + c

    zeros16 = jnp.zeros((16,), jnp.float32)

    def zrow(r, carry):
        for j in range(ACCW // 16):
            msg_v[r, pl.ds(j * 16, 16)] = zeros16
        return carry

    lax.fori_loop(0, EB, zrow, 0)
    base_r = s * RPT
    nfull = RPT // EB
    for k in range(nfull):
        pltpu.sync_copy(msg_v, acc.at[pl.ds(base_r + k * EB, EB)])
    rem = RPT - nfull * EB
    if rem:
        pltpu.sync_copy(msg_v.at[pl.ds(0, rem)],
                        acc.at[pl.ds(base_r + nfull * EB, rem)])
    plsc.subcore_barrier()

    def issue(ch, p):
        base = wid * EPW + ch * EB
        pltpu.sync_copy(ei_hbm.at[:, pl.ds(base, EB)], sd_v[p])
        pltpu.async_copy(asd_hbm.at[sd_v[p].at[0]], as_v[p], sems[p])
        pltpu.async_copy(dsa_hbm.at[sd_v[p].at[1]], ad_v[p], sems[p])
        pltpu.async_copy(h_hbm.at[sd_v[p].at[0]], h_v[p], sems[p])

    def drain(p):
        pltpu.make_async_copy(asd_hbm.at[sd_v[p].at[0]], as_v[p], sems[p]).wait()
        pltpu.make_async_copy(dsa_hbm.at[sd_v[p].at[1]], ad_v[p], sems[p]).wait()
        pltpu.make_async_copy(h_hbm.at[sd_v[p].at[0]], h_v[p], sems[p]).wait()

    def compute(p):
        @plsc.parallel_loop(0, EB, 1, unroll=8)
        def edge(bi):
            sv = as_v[p][bi, :] + ad_v[p][bi, :]
            w = jnp.exp(jnp.maximum(sv, 0.2 * sv))
            msg_v[bi, pl.ds(D, 16)] = w
            for hh in range(HEADS):
                wsp = lax.gather(
                    w, jnp.full((16, 1), hh, jnp.int32),
                    lax.GatherDimensionNumbers(
                        offset_dims=(), collapsed_slice_dims=(0,),
                        start_index_map=(0,)),
                    (1,), mode=lax.GatherScatterMode.PROMISE_IN_BOUNDS)
                msg_v[bi, pl.ds(hh * 16, 16)] = (
                    h_v[p][bi, pl.ds(hh * 16, 16)] * wsp)

        pltpu.sync_copy(msg_v, acc.at[sd_v[p].at[1]], add=True)

    issue(0, 0)

    def chunk2(i, carry):
        ch = 2 * i
        issue(ch + 1, 1)
        drain(0)
        compute(0)
        issue(ch + 2, 0)
        drain(1)
        compute(1)
        return carry

    lax.fori_loop(0, (NCHUNK - 1) // 2, chunk2, 0)
    drain(0)
    compute(0)
    plsc.subcore_barrier()

    for k in range(nfull):
        rows = pl.ds(base_r + k * EB, EB)
        pltpu.sync_copy(acc.at[rows], out_hbm.at[c, rows])
    if rem:
        rows = pl.ds(base_r + nfull * EB, rem)
        pltpu.sync_copy(acc.at[rows], out_hbm.at[c, rows])


@functools.lru_cache(maxsize=1)
def _make_edge_kernel():
    return pl.kernel(
        _edge_body,
        out_type=jax.ShapeDtypeStruct((NCORES, N, ACCW), jnp.float32),
        mesh=plsc.VectorSubcoreMesh(core_axis_name="c", subcore_axis_name="s",
                                    num_cores=NCORES, num_subcores=NSUB),
        compiler_params=pltpu.CompilerParams(use_tc_tiling_on_sc=False),
        scratch_types=[
            pltpu.VMEM((2, EB), jnp.int32),
            pltpu.VMEM((2, EB), jnp.int32),
            pltpu.VMEM((EB, 16), jnp.float32),
            pltpu.VMEM((EB, 16), jnp.float32),
            pltpu.VMEM((EB, 16), jnp.float32),
            pltpu.VMEM((EB, 16), jnp.float32),
            pltpu.VMEM((EB, D), jnp.float32),
            pltpu.VMEM((EB, D), jnp.float32),
            pltpu.VMEM((EB, ACCW), jnp.float32),
            pltpu.VMEM_SHARED((N, ACCW), jnp.float32),
            pltpu.SemaphoreType.DMA,
            pltpu.SemaphoreType.DMA,
        ],
    )


def _edge_pass(edge_index, asd, h):
    dsa = jnp.concatenate([asd[:, HEADS:], asd[:, :HEADS]], axis=1)
    acc = _make_edge_kernel()(edge_index, asd, dsa, h)
    return acc[:, :, :D], acc[:, :, D:]


def _ascat(a_src, a_dst):
    mask = (jnp.arange(D)[:, None] // HID == jnp.arange(HEADS)[None, :])
    asm = jnp.where(mask, a_src.reshape(D)[:, None], 0.0)
    adm = jnp.where(mask, a_dst.reshape(D)[:, None], 0.0)
    return jnp.concatenate([asm, adm], axis=1)


def kernel(x, edge_index, W1, a_src1, a_dst1, b1, ln1_g, ln1_b,
           W2, a_src2, a_dst2, b2, ln2_g, ln2_b, W_out, b_out):
    exp8 = (jnp.arange(HEADS)[:, None] == jnp.arange(D)[None, :] // HID)
    exp8 = exp8.astype(jnp.float32)

    h1, asd1 = _dense_in(x, W1, _ascat(a_src1, a_dst1))
    msgs1, dens1 = _edge_pass(edge_index, asd1, h1)
    h2, asd2 = _dense_mid(msgs1, dens1, h1, asd1, exp8,
                          b1.reshape(1, D), ln1_g.reshape(1, D),
                          ln1_b.reshape(1, D), W2, _ascat(a_src2, a_dst2))
    msgs2, dens2 = _edge_pass(edge_index, asd2, h2)
    out, = _dense_out(msgs2, dens2, h2, asd2, exp8,
                      b2.reshape(1, D), ln2_g.reshape(1, D),
                      ln2_b.reshape(1, D), W_out, b_out.reshape(1, NCLS))
    return out


# speedup vs baseline: 2.9820x; 1.0979x over previous
import functools

import jax
import jax.numpy as jnp
from jax import lax
from jax.experimental import pallas as pl
from jax.experimental.pallas import tpu as pltpu
from jax.experimental.pallas import tpu_sc as plsc

N = 10000
E = 320000
D = 128
HEADS = 8
HID = 16
NCLS = 64

NCORES = 2
NSUB = 16
NW = NCORES * NSUB
EPW = E // NW
EB = 80
NCHUNK = EPW // EB
ACCW = D + 16
RPT = N // NSUB
RZ = 125

ROWB = 1000
GRID = N // ROWB


def _dense_in_body(x_ref, w_ref, ascat_ref, h_ref, asd_ref, dsa_ref):
    h = jnp.dot(x_ref[...], w_ref[...], preferred_element_type=jnp.float32)
    h_ref[...] = h
    asd = jnp.dot(h, ascat_ref[...], preferred_element_type=jnp.float32)
    asd_ref[...] = asd
    dsa_ref[...] = jnp.concatenate([asd[:, HEADS:], asd[:, :HEADS]], axis=1)


def _dense_in(x, w, ascat):
    return pl.pallas_call(
        _dense_in_body,
        grid=(GRID,),
        in_specs=[
            pl.BlockSpec((ROWB, D), lambda i: (i, 0)),
            pl.BlockSpec((D, D), lambda i: (0, 0)),
            pl.BlockSpec((D, 16), lambda i: (0, 0)),
        ],
        out_specs=[
            pl.BlockSpec((ROWB, D), lambda i: (i, 0)),
            pl.BlockSpec((ROWB, 16), lambda i: (i, 0)),
            pl.BlockSpec((ROWB, 16), lambda i: (i, 0)),
        ],
        out_shape=[
            jax.ShapeDtypeStruct((N, D), jnp.float32),
            jax.ShapeDtypeStruct((N, 16), jnp.float32),
            jax.ShapeDtypeStruct((N, 16), jnp.float32),
        ],
    )(x, w, ascat)


def _combine(accb, h, asd, exp8, bias, g, b):
    comb = accb[0] + accb[1]
    comb_m = comb[:, :D]
    d8 = comb[:, D:D + HEADS]
    ss = asd[:, :HEADS] + asd[:, HEADS:2 * HEADS]
    wself = jnp.exp(jnp.maximum(ss, 0.2 * ss))
    wself128 = jnp.dot(wself, exp8, preferred_element_type=jnp.float32)
    num = comb_m + wself128 * h
    den = jnp.dot(d8 + wself, exp8, preferred_element_type=jnp.float32)
    o = num / (den + 1e-16) + bias
    m = o.mean(-1, keepdims=True)
    v = ((o - m) ** 2).mean(-1, keepdims=True)
    return jax.nn.relu((o - m) / jnp.sqrt(v + 1e-5) * g + b)


def _dense_mid_body(acc_ref, h_ref, asd_ref, exp8_ref, b1_ref,
                    g_ref, bln_ref, w2_ref, ascat2_ref,
                    h2_ref, asd2_ref, dsa2_ref):
    y = _combine(acc_ref[...], h_ref[...], asd_ref[...],
                 exp8_ref[...], b1_ref[...], g_ref[...], bln_ref[...])
    h2 = jnp.dot(y, w2_ref[...], preferred_element_type=jnp.float32)
    h2_ref[...] = h2
    asd2 = jnp.dot(h2, ascat2_ref[...], preferred_element_type=jnp.float32)
    asd2_ref[...] = asd2
    dsa2_ref[...] = jnp.concatenate([asd2[:, HEADS:], asd2[:, :HEADS]], axis=1)


def _dense_mid(acc, h, asd, exp8, b1, g, bln, w2, ascat2):
    return pl.pallas_call(
        _dense_mid_body,
        grid=(GRID,),
        in_specs=[
            pl.BlockSpec((2, ROWB, ACCW), lambda i: (0, i, 0)),
            pl.BlockSpec((ROWB, D), lambda i: (i, 0)),
            pl.BlockSpec((ROWB, 16), lambda i: (i, 0)),
            pl.BlockSpec((HEADS, D), lambda i: (0, 0)),
            pl.BlockSpec((1, D), lambda i: (0, 0)),
            pl.BlockSpec((1, D), lambda i: (0, 0)),
            pl.BlockSpec((1, D), lambda i: (0, 0)),
            pl.BlockSpec((D, D), lambda i: (0, 0)),
            pl.BlockSpec((D, 16), lambda i: (0, 0)),
        ],
        out_specs=[
            pl.BlockSpec((ROWB, D), lambda i: (i, 0)),
            pl.BlockSpec((ROWB, 16), lambda i: (i, 0)),
            pl.BlockSpec((ROWB, 16), lambda i: (i, 0)),
        ],
        out_shape=[
            jax.ShapeDtypeStruct((N, D), jnp.float32),
            jax.ShapeDtypeStruct((N, 16), jnp.float32),
            jax.ShapeDtypeStruct((N, 16), jnp.float32),
        ],
    )(acc, h, asd, exp8, b1, g, bln, w2, ascat2)


def _dense_out_body(acc_ref, h_ref, asd_ref, exp8_ref, b2_ref,
                    g_ref, bln_ref, wout_ref, bout_ref, out_ref):
    y = _combine(acc_ref[...], h_ref[...], asd_ref[...],
                 exp8_ref[...], b2_ref[...], g_ref[...], bln_ref[...])
    out_ref[...] = jnp.dot(y, wout_ref[...],
                           preferred_element_type=jnp.float32) + bout_ref[...]


def _dense_out(acc, h, asd, exp8, b2, g, bln, wout, bout):
    return pl.pallas_call(
        _dense_out_body,
        grid=(GRID,),
        in_specs=[
            pl.BlockSpec((2, ROWB, ACCW), lambda i: (0, i, 0)),
            pl.BlockSpec((ROWB, D), lambda i: (i, 0)),
            pl.BlockSpec((ROWB, 16), lambda i: (i, 0)),
            pl.BlockSpec((HEADS, D), lambda i: (0, 0)),
            pl.BlockSpec((1, D), lambda i: (0, 0)),
            pl.BlockSpec((1, D), lambda i: (0, 0)),
            pl.BlockSpec((1, D), lambda i: (0, 0)),
            pl.BlockSpec((D, NCLS), lambda i: (0, 0)),
            pl.BlockSpec((1, NCLS), lambda i: (0, 0)),
        ],
        out_specs=[pl.BlockSpec((ROWB, NCLS), lambda i: (i, 0))],
        out_shape=[jax.ShapeDtypeStruct((N, NCLS), jnp.float32)],
    )(acc, h, asd, exp8, b2, g, bln, wout, bout)


def _edge_body(ei_hbm, asd_hbm, dsa_hbm, h_hbm, out_hbm,
               sd_v0, sd_v1, as_v0, as_v1, ad_v0, ad_v1,
               h_v0, h_v1, msg_v, acc, sem0, sem1):
    sd_v = (sd_v0, sd_v1)
    as_v = (as_v0, as_v1)
    ad_v = (ad_v0, ad_v1)
    h_v = (h_v0, h_v1)
    sems = (sem0, sem1)
    c = lax.axis_index("c")
    s = lax.axis_index("s")
    wid = s * NCORES + c

    zeros16 = jnp.zeros((16,), jnp.float32)

    def zrow(r, carry):
        for j in range(ACCW // 16):
            msg_v[r, pl.ds(j * 16, 16)] = zeros16
        return carry

    lax.fori_loop(0, EB, zrow, 0)
    base_r = s * RPT
    nfull = RPT // EB
    for k in range(nfull):
        pltpu.sync_copy(msg_v, acc.at[pl.ds(base_r + k * EB, EB)])
    rem = RPT - nfull * EB
    if rem:
        pltpu.sync_copy(msg_v.at[pl.ds(0, rem)],
                        acc.at[pl.ds(base_r + nfull * EB, rem)])
    plsc.subcore_barrier()

    def issue(ch, p):
        base = wid * EPW + ch * EB
        pltpu.sync_copy(ei_hbm.at[:, pl.ds(base, EB)], sd_v[p])
        pltpu.async_copy(asd_hbm.at[sd_v[p].at[0]], as_v[p], sems[p])
        pltpu.async_copy(dsa_hbm.at[sd_v[p].at[1]], ad_v[p], sems[p])
        pltpu.async_copy(h_hbm.at[sd_v[p].at[0]], h_v[p], sems[p])

    def drain(p):
        pltpu.make_async_copy(asd_hbm.at[sd_v[p].at[0]], as_v[p], sems[p]).wait()
        pltpu.make_async_copy(dsa_hbm.at[sd_v[p].at[1]], ad_v[p], sems[p]).wait()
        pltpu.make_async_copy(h_hbm.at[sd_v[p].at[0]], h_v[p], sems[p]).wait()

    def compute(p):
        @plsc.parallel_loop(0, EB, 1, unroll=8)
        def edge(bi):
            sv = as_v[p][bi, :] + ad_v[p][bi, :]
            w = jnp.exp(jnp.maximum(sv, 0.2 * sv))
            msg_v[bi, pl.ds(D, 16)] = w
            for hh in range(HEADS):
                wsp = lax.gather(
                    w, jnp.full((16, 1), hh, jnp.int32),
                    lax.GatherDimensionNumbers(
                        offset_dims=(), collapsed_slice_dims=(0,),
                        start_index_map=(0,)),
                    (1,), mode=lax.GatherScatterMode.PROMISE_IN_BOUNDS)
                msg_v[bi, pl.ds(hh * 16, 16)] = (
                    h_v[p][bi, pl.ds(hh * 16, 16)] * wsp)

        pltpu.sync_copy(msg_v, acc.at[sd_v[p].at[1]], add=True)

    issue(0, 0)

    def chunk2(i, carry):
        ch = 2 * i
        issue(ch + 1, 1)
        drain(0)
        compute(0)
        issue(ch + 2, 0)
        drain(1)
        compute(1)
        return carry

    lax.fori_loop(0, (NCHUNK - 1) // 2, chunk2, 0)
    drain(0)
    compute(0)
    plsc.subcore_barrier()

    for k in range(nfull):
        rows = pl.ds(base_r + k * EB, EB)
        pltpu.sync_copy(acc.at[rows], out_hbm.at[c, rows])
    if rem:
        rows = pl.ds(base_r + nfull * EB, rem)
        pltpu.sync_copy(acc.at[rows], out_hbm.at[c, rows])


@functools.lru_cache(maxsize=1)
def _make_edge_kernel():
    return pl.kernel(
        _edge_body,
        out_type=jax.ShapeDtypeStruct((NCORES, N, ACCW), jnp.float32),
        mesh=plsc.VectorSubcoreMesh(core_axis_name="c", subcore_axis_name="s",
                                    num_cores=NCORES, num_subcores=NSUB),
        compiler_params=pltpu.CompilerParams(use_tc_tiling_on_sc=False),
        scratch_types=[
            pltpu.VMEM((2, EB), jnp.int32),
            pltpu.VMEM((2, EB), jnp.int32),
            pltpu.VMEM((EB, 16), jnp.float32),
            pltpu.VMEM((EB, 16), jnp.float32),
            pltpu.VMEM((EB, 16), jnp.float32),
            pltpu.VMEM((EB, 16), jnp.float32),
            pltpu.VMEM((EB, D), jnp.float32),
            pltpu.VMEM((EB, D), jnp.float32),
            pltpu.VMEM((EB, ACCW), jnp.float32),
            pltpu.VMEM_SHARED((N, ACCW), jnp.float32),
            pltpu.SemaphoreType.DMA,
            pltpu.SemaphoreType.DMA,
        ],
    )


def _edge_pass(edge_index, asd, dsa, h):
    return _make_edge_kernel()(edge_index, asd, dsa, h)


def _ascat(a_src, a_dst):
    mask = (jnp.arange(D)[:, None] // HID == jnp.arange(HEADS)[None, :])
    asm = jnp.where(mask, a_src.reshape(D)[:, None], 0.0)
    adm = jnp.where(mask, a_dst.reshape(D)[:, None], 0.0)
    return jnp.concatenate([asm, adm], axis=1)


def kernel(x, edge_index, W1, a_src1, a_dst1, b1, ln1_g, ln1_b,
           W2, a_src2, a_dst2, b2, ln2_g, ln2_b, W_out, b_out):
    exp8 = (jnp.arange(HEADS)[:, None] == jnp.arange(D)[None, :] // HID)
    exp8 = exp8.astype(jnp.float32)

    h1, asd1, dsa1 = _dense_in(x, W1, _ascat(a_src1, a_dst1))
    acc1 = _edge_pass(edge_index, asd1, dsa1, h1)
    h2, asd2, dsa2 = _dense_mid(acc1, h1, asd1, exp8,
                                b1.reshape(1, D), ln1_g.reshape(1, D),
                                ln1_b.reshape(1, D), W2, _ascat(a_src2, a_dst2))
    acc2 = _edge_pass(edge_index, asd2, dsa2, h2)
    out, = _dense_out(acc2, h2, asd2, exp8,
                      b2.reshape(1, D), ln2_g.reshape(1, D),
                      ln2_b.reshape(1, D), W_out, b_out.reshape(1, NCLS))
    return out
